# baseline (speedup 1.0000x reference)
import jax
import jax.numpy as jnp
from jax.experimental import pallas as pl
from jax.experimental.pallas import tpu as pltpu

_M = 4
_N = 8
_K = 2 * 768 * 2048
_S = 4
_KS = _K // _S
_CK = 65536
_T = _KS // _CK


def _gate_body(*refs):
    x_refs = refs[:_S]
    w_refs = refs[_S:2 * _S]
    b_ref = refs[2 * _S]
    o_ref = refs[2 * _S + 1]
    acc_ref = refs[2 * _S + 2]

    @pl.when(pl.program_id(0) == 0)
    def _init():
        acc_ref[...] = jnp.zeros_like(acc_ref)

    p = jnp.zeros((_M, _N), jnp.float32)
    for s in range(_S):
        p += jax.lax.dot_general(
            x_refs[s][:, 0, 0, :], w_refs[s][:, 0, 0, :],
            dimension_numbers=(((1,), (1,)), ((), ())),
            preferred_element_type=jnp.float32,
        )
    acc_ref[...] += p

    @pl.when(pl.program_id(0) == _T - 1)
    def _finish():
        v = acc_ref[...] + b_ref[...]
        lane = jax.lax.broadcasted_iota(jnp.int32, v.shape, 1)

        def drop_one_max(u):
            m = jnp.max(u, axis=1, keepdims=True)
            first = jnp.min(
                jnp.where(u == m, lane, v.shape[1]), axis=1, keepdims=True
            )
            return jnp.where(lane == first, -jnp.inf, u)

        thr = jnp.max(drop_one_max(drop_one_max(v)), axis=1, keepdims=True)
        m1 = jnp.max(v, axis=1, keepdims=True)
        e = jnp.where(v > thr, jnp.exp(v - m1), 0.0)
        o_ref[...] = e / jnp.sum(e, axis=1, keepdims=True)


def _x_spec(s):
    return pl.BlockSpec((_M, 1, 1, _CK), lambda i, s=s: (0, s, 0, i))


def _w_spec(s):
    return pl.BlockSpec((_N, 1, 1, _CK), lambda i, s=s: (0, s, 0, i))


@jax.jit
def kernel(x, W, b):
    xr = x.reshape(_M, _S, 1, _KS)
    wr = W.reshape(_N, _S, 1, _KS)
    return pl.pallas_call(
        _gate_body,
        grid=(_T,),
        in_specs=(
            [_x_spec(s) for s in range(_S)]
            + [_w_spec(s) for s in range(_S)]
            + [pl.BlockSpec((1, _N), lambda i: (0, 0))]
        ),
        out_specs=pl.BlockSpec((_M, _N), lambda i: (0, 0)),
        out_shape=jax.ShapeDtypeStruct((_M, _N), jnp.float32),
        scratch_shapes=[pltpu.VMEM((_M, _N), jnp.float32)],
        compiler_params=pltpu.CompilerParams(
            dimension_semantics=("arbitrary",),
        ),
    )(*([xr] * _S + [wr] * _S + [b.reshape(1, -1)]))


# speedup vs baseline: 5.6195x; 5.6195x over previous
import jax
import jax.numpy as jnp
from jax.experimental import pallas as pl
from jax.experimental.pallas import tpu as pltpu

_M = 4
_N = 8
_K = 2 * 768 * 2048
_S = 4
_CK = 65536
_T = _K // (_S * _CK)


def _gate_body(*refs):
    x_refs = refs[:_S]
    w_refs = refs[_S:2 * _S]
    b_ref = refs[2 * _S]
    o_ref = refs[2 * _S + 1]
    acc_ref = refs[2 * _S + 2]

    @pl.when(pl.program_id(0) == 0)
    def _init():
        acc_ref[...] = jnp.zeros_like(acc_ref)

    p = jnp.zeros((_M, _N), jnp.float32)
    for s in range(_S):
        p += jax.lax.dot_general(
            x_refs[s][...], w_refs[s][...],
            dimension_numbers=(((1,), (1,)), ((), ())),
            preferred_element_type=jnp.float32,
        )
    acc_ref[...] += p

    @pl.when(pl.program_id(0) == _T - 1)
    def _finish():
        v = acc_ref[...] + b_ref[...]
        lane = jax.lax.broadcasted_iota(jnp.int32, v.shape, 1)

        def drop_one_max(u):
            m = jnp.max(u, axis=1, keepdims=True)
            first = jnp.min(
                jnp.where(u == m, lane, v.shape[1]), axis=1, keepdims=True
            )
            return jnp.where(lane == first, -jnp.inf, u)

        thr = jnp.max(drop_one_max(drop_one_max(v)), axis=1, keepdims=True)
        m1 = jnp.max(v, axis=1, keepdims=True)
        e = jnp.where(v > thr, jnp.exp(v - m1), 0.0)
        o_ref[...] = e / jnp.sum(e, axis=1, keepdims=True)


def _x_spec(s):
    return pl.BlockSpec((_M, _CK), lambda i, s=s: (0, s * _T + i))


def _w_spec(s):
    return pl.BlockSpec((_N, _CK), lambda i, s=s: (0, s * _T + i))


@jax.jit
def kernel(x, W, b):
    xf = x.reshape(_M, _K)
    return pl.pallas_call(
        _gate_body,
        grid=(_T,),
        in_specs=(
            [_x_spec(s) for s in range(_S)]
            + [_w_spec(s) for s in range(_S)]
            + [pl.BlockSpec((1, _N), lambda i: (0, 0))]
        ),
        out_specs=pl.BlockSpec((_M, _N), lambda i: (0, 0)),
        out_shape=jax.ShapeDtypeStruct((_M, _N), jnp.float32),
        scratch_shapes=[pltpu.VMEM((_M, _N), jnp.float32)],
        compiler_params=pltpu.CompilerParams(
            dimension_semantics=("arbitrary",),
        ),
    )(*([xf] * _S + [W] * _S + [b.reshape(1, -1)]))


# speedup vs baseline: 12.1883x; 2.1689x over previous
import jax
import jax.numpy as jnp
from jax.experimental import pallas as pl
from jax.experimental.pallas import tpu as pltpu

_M = 4
_N = 8
_SEQ = 2048
_D = 1536
_K = _SEQ * _D
_R = 128
_T = _SEQ // _R


def _gate_body(x_ref, w_ref, b_ref, o_ref, acc_ref):
    @pl.when(pl.program_id(0) == 0)
    def _init():
        acc_ref[...] = jnp.zeros_like(acc_ref)

    p = jnp.zeros((_M, _N), jnp.float32)
    for j in range(_R):
        p += jax.lax.dot_general(
            x_ref[:, j, :], w_ref[:, j * _D:(j + 1) * _D],
            dimension_numbers=(((1,), (1,)), ((), ())),
            preferred_element_type=jnp.float32,
        )
    acc_ref[...] += p

    @pl.when(pl.program_id(0) == _T - 1)
    def _finish():
        v = acc_ref[...] + b_ref[...]
        lane = jax.lax.broadcasted_iota(jnp.int32, v.shape, 1)

        def drop_one_max(u):
            m = jnp.max(u, axis=1, keepdims=True)
            first = jnp.min(
                jnp.where(u == m, lane, v.shape[1]), axis=1, keepdims=True
            )
            return jnp.where(lane == first, -jnp.inf, u)

        thr = jnp.max(drop_one_max(drop_one_max(v)), axis=1, keepdims=True)
        m1 = jnp.max(v, axis=1, keepdims=True)
        e = jnp.where(v > thr, jnp.exp(v - m1), 0.0)
        o_ref[...] = e / jnp.sum(e, axis=1, keepdims=True)


@jax.jit
def kernel(x, W, b):
    return pl.pallas_call(
        _gate_body,
        grid=(_T,),
        in_specs=[
            pl.BlockSpec((_M, _R, _D), lambda i: (0, i, 0)),
            pl.BlockSpec((_N, _R * _D), lambda i: (0, i)),
            pl.BlockSpec((1, _N), lambda i: (0, 0)),
        ],
        out_specs=pl.BlockSpec((_M, _N), lambda i: (0, 0)),
        out_shape=jax.ShapeDtypeStruct((_M, _N), jnp.float32),
        scratch_shapes=[pltpu.VMEM((_M, _N), jnp.float32)],
        compiler_params=pltpu.CompilerParams(
            dimension_semantics=("arbitrary",),
        ),
    )(x, W, b.reshape(1, -1))
